# baseline (speedup 1.0000x reference)
import functools

import jax
import jax.numpy as jnp
from jax import lax
from jax.experimental import pallas as pl
from jax.experimental.pallas import tpu as pltpu
from jax.experimental.pallas import tpu_sc as plsc

C = 64
NX, NY = 432, 496
NPOS = NX * NY
P = 4512
L = 16
NC, NS = 2, 16
NW = NC * NS
BLK = 256
NBLK = NPOS // BLK
BPW = -(-NBLK // NW)
PV = P // L
PID_BITS = 13
PID_MASK = (1 << PID_BITS) - 1
KEY_SENT = 1 << (PID_BITS + 8)

_MESH = plsc.VectorSubcoreMesh(
    core_axis_name="c", subcore_axis_name="s", num_cores=NC, num_subcores=NS
)


@functools.partial(
    pl.kernel,
    out_type=jax.ShapeDtypeStruct((C, NPOS), jnp.float32),
    mesh=_MESH,
    scratch_types=[
        pltpu.VMEM((P * 4,), jnp.int32),
        pltpu.VMEM((P + 2 * L,), jnp.int32),
        pltpu.VMEM((P + 2 * L,), jnp.int32),
        pltpu.VMEM((C, BLK), jnp.float32),
        pltpu.VMEM((L, 2 * C), jnp.float32),
        pltpu.VMEM((2 * L,), jnp.int32),
        pltpu.SemaphoreType.DMA,
    ],
    compiler_params=pltpu.CompilerParams(needs_layout_passes=False),
)
def _scatter_kernel(pf_hbm, coords_hbm, out_hbm,
                    coords_v, keyw, keyb, slab, rowbuf, nxtb, sem_row):
    w = lax.axis_index("s") * NC + lax.axis_index("c")
    iota = lax.iota(jnp.int32, L)
    zeros16 = jnp.zeros((L,), jnp.float32)

    pltpu.sync_copy(coords_hbm, coords_v)

    def scan_body(i, cnt):
        p0 = i * L
        base4 = (p0 + iota) * 4
        c1 = plsc.load_gather(coords_v, [base4 + 1])
        c2 = plsc.load_gather(coords_v, [base4 + 2])
        c3 = plsc.load_gather(coords_v, [base4 + 3])
        idx = c1 + c2 * NX + c3
        key = idx * (1 << PID_BITS) + (p0 + iota)
        m = ((idx >> 8) & (NW - 1)) == w
        mi = m.astype(jnp.int32)
        pos = cnt + plsc.cumsum(mi) - 1
        pos = jnp.where(m, pos, 0)
        plsc.store_scatter(keyw, [pos], key, mask=m)
        return cnt + jnp.sum(mi)

    nmemb = lax.fori_loop(0, PV, scan_body, jnp.int32(0))
    plsc.store_scatter(keyw, [nmemb + iota], jnp.full((L,), -1, jnp.int32))

    def zrow(r, _):
        for q in range(BLK // L):
            slab[r, pl.ds(q * L, L)] = zeros16
        return 0
    lax.fori_loop(0, C, zrow, 0)
    nxtb[pl.ds(L, L)] = jnp.full((L,), KEY_SENT, jnp.int32)

    def block_body(k, _):
        blk = w + NW * k

        @pl.when(blk < NBLK)
        def _():
            base = blk * BLK

            def fbody(i, cnt):
                kv = keyw[pl.ds(i * L, L)]
                mb = (kv >> (PID_BITS + 8)) == blk
                mi = mb.astype(jnp.int32)
                pos = cnt + plsc.cumsum(mi) - 1
                pos = jnp.where(mb, pos, 0)
                plsc.store_scatter(keyb, [pos], kv & (KEY_SENT - 1), mask=mb)
                return cnt + jnp.sum(mi)

            nloc = lax.fori_loop(0, (nmemb + L - 1) // L, fbody, jnp.int32(0))
            plsc.store_scatter(keyb, [nloc + iota],
                               jnp.full((L,), KEY_SENT, jnp.int32))

            def gbody(g, _):
                kv = keyb[pl.ds(g * L, L)]
                sk, _sv = plsc.sort_key_val(kv, kv)
                nxtb[pl.ds(0, L)] = sk
                nxt = nxtb[pl.ds(1, L)]
                win = ((sk >> PID_BITS) != (nxt >> PID_BITS)) & (sk < KEY_SENT)
                col = (sk >> PID_BITS) & (BLK - 1)
                pid = sk & PID_MASK
                pltpu.async_copy(pf_hbm.at[pid >> 1], rowbuf, sem_row).wait()
                half = (pid & 1) * C
                for c in range(C):
                    cc = jnp.full((L,), c, jnp.int32)
                    src = plsc.load_gather(rowbuf, [iota, half + c])
                    plsc.store_scatter(slab, [cc, col], src, mask=win)
                return 0

            ngrp = (nloc + L - 1) // L
            lax.fori_loop(0, ngrp, gbody, 0)

            pltpu.sync_copy(slab, out_hbm.at[:, pl.ds(base, BLK)])

            def zbody(g, _):
                kv = keyb[pl.ds(g * L, L)]
                mb = kv < KEY_SENT
                colz = (kv >> PID_BITS) & (BLK - 1)
                for c in range(C):
                    cc = jnp.full((L,), c, jnp.int32)
                    plsc.store_scatter(slab, [cc, colz], zeros16, mask=mb)
                return 0

            lax.fori_loop(0, ngrp, zbody, 0)
        return 0

    lax.fori_loop(0, BPW, block_body, 0)


def kernel(pillar_features, coords):
    coords_flat = coords.reshape(P * 4).astype(jnp.int32)
    pf_pairs = pillar_features.reshape(P // 2, 2 * C)
    canvas = _scatter_kernel(pf_pairs, coords_flat)
    return canvas.reshape(1, C, NY, NX)
